# baseline (speedup 1.0000x reference)
import jax
import jax.numpy as jnp
from jax.experimental import pallas as pl

_BLK_S = 256


def _pe_add_kernel(x_ref, emb_ref, o_ref):
    e = emb_ref[...]
    b = x_ref.shape[1] // e.shape[1]
    o_ref[...] = x_ref[...] + jnp.concatenate([e] * b, axis=1)


def kernel(x, emb):
    S, B, D = x.shape
    x2 = x.reshape(S, B * D)
    grid = (S // _BLK_S,)
    out = pl.pallas_call(
        _pe_add_kernel,
        grid=grid,
        in_specs=[
            pl.BlockSpec((_BLK_S, B * D), lambda i: (i, 0)),
            pl.BlockSpec((_BLK_S, D), lambda i: (i, 0)),
        ],
        out_specs=pl.BlockSpec((_BLK_S, B * D), lambda i: (i, 0)),
        out_shape=jax.ShapeDtypeStruct((S, B * D), x.dtype),
    )(x2, emb)
    return out.reshape(S, B, D)


# speedup vs baseline: 4.2043x; 4.2043x over previous
import jax
import jax.numpy as jnp
from jax.experimental import pallas as pl

_BLK_S = 512


def _pe_add_kernel(x_ref, emb_ref, o_ref):
    o_ref[...] = x_ref[...] + emb_ref[...][:, None, :]


def kernel(x, emb):
    S, B, D = x.shape
    grid = (S // _BLK_S,)
    return pl.pallas_call(
        _pe_add_kernel,
        grid=grid,
        in_specs=[
            pl.BlockSpec((_BLK_S, B, D), lambda i: (i, 0, 0)),
            pl.BlockSpec((_BLK_S, D), lambda i: (i, 0)),
        ],
        out_specs=pl.BlockSpec((_BLK_S, B, D), lambda i: (i, 0, 0)),
        out_shape=jax.ShapeDtypeStruct((S, B, D), x.dtype),
    )(x, emb)
